# Initial kernel scaffold; baseline (speedup 1.0000x reference)
import jax
import jax.numpy as jnp
from jax.experimental import pallas as pl


def kernel(u_embs, i_embs, edge_index):
    raise NotImplementedError("write your pallas kernel here")


# speedup vs baseline: 12.5933x; 12.5933x over previous
import functools

import jax
import jax.numpy as jnp
from jax import lax
from jax.experimental import pallas as pl
from jax.experimental.pallas import tpu as pltpu
from jax.experimental.pallas import tpu_sc as plsc

N_USERS = 5000
N_ITEMS = 5000
N = N_USERS + N_ITEMS
D = 128
NPAD = 10240
TRASH = 10100
E2 = 640000
NC, NS = 2, 16
NW = NC * NS
C = 128
NCB = 32
NBK = 5
NCH = NCB * NBK
EPT = NCH * C
RPT = NPAD // NW
RPS = NPAD // NS

_f32 = jnp.float32


def _wid():
    return lax.axis_index("c") * NS + lax.axis_index("s")


def _deg_body(rows_hbm, zdeg_hbm, deg_hbm, rows_v, ones_v, deg_sh):
    c = lax.axis_index("c")
    s = lax.axis_index("s")
    pltpu.sync_copy(zdeg_hbm.at[pl.ds(s * RPS, RPS)],
                    deg_sh.at[pl.ds(s * RPS, RPS)])
    for k in range(C // 16):
        ones_v[pl.ds(k * 16, 16)] = jnp.full((16,), 1.0, _f32)
    pltpu.sync_copy(rows_hbm.at[_wid()], rows_v)
    plsc.subcore_barrier()

    def step(j, carry):
        pltpu.sync_copy(ones_v, deg_sh.at[rows_v.at[j]], add=True)
        return carry

    lax.fori_loop(0, NCH, step, 0)
    plsc.subcore_barrier()
    pltpu.sync_copy(deg_sh.at[pl.ds(s * RPS, RPS)],
                    deg_hbm.at[c, pl.ds(s * RPS, RPS)])


def _isd_body(deg_ref, isd_ref):
    d = deg_ref[0] + deg_ref[1]
    isd_ref[...] = jnp.where(
        d > 0.0, lax.rsqrt(jnp.maximum(d, 1.0)), 0.0)


def _isd_tc(deg2):
    return pl.pallas_call(
        _isd_body,
        out_shape=jax.ShapeDtypeStruct((NPAD // D, D), _f32),
    )(deg2.reshape(NC, NPAD // D, D))


def _edge_body(xs_hbm, rows_hbm, cols_hbm, z2d_hbm, p_hbm,
               rows_v, cols_v, buf, accum, sem):
    c = lax.axis_index("c")
    s = lax.axis_index("s")
    w = _wid()
    pltpu.sync_copy(z2d_hbm, accum.at[pl.ds(s * RPS, RPS)])
    plsc.subcore_barrier()

    def blk(bi, carry):
        pltpu.sync_copy(rows_hbm.at[w, pl.ds(bi * NCB, NCB)], rows_v)
        pltpu.sync_copy(cols_hbm.at[w, pl.ds(bi * NCB, NCB)], cols_v)

        def step(j, carry2):
            pltpu.async_copy(xs_hbm.at[cols_v.at[j]], buf, sem).wait()
            pltpu.sync_copy(buf, accum.at[rows_v.at[j]], add=True)
            return carry2

        lax.fori_loop(0, NCB, step, 0)
        return carry

    lax.fori_loop(0, NBK, blk, 0)
    plsc.subcore_barrier()
    pltpu.sync_copy(accum.at[pl.ds(s * RPS, RPS)],
                    p_hbm.at[c, pl.ds(s * RPS, RPS)])


def _scale_rows(isd_v, fn):
    def grp(g, carry):
        base = pl.multiple_of(g * 16, 16)
        sv16 = isd_v[pl.ds(base, 16)]
        for i in range(16):
            fn(base + i, jnp.full((16,), sv16[i], _f32))
        return carry
    lax.fori_loop(0, RPT // 16, grp, 0)


def _scale0_body(x_hbm, isd_hbm, xs_hbm, buf, isd_v):
    base = _wid() * RPT
    pltpu.sync_copy(x_hbm.at[pl.ds(base, RPT)], buf)
    pltpu.sync_copy(isd_hbm.at[pl.ds(base, RPT)], isd_v)

    def row(r, svec):
        for k in range(D // 16):
            sl = pl.ds(k * 16, 16)
            buf[r, sl] = buf[r, sl] * svec

    _scale_rows(isd_v, row)
    pltpu.sync_copy(buf, xs_hbm.at[pl.ds(base, RPT)])


def _combine1_body(p_hbm, isd_hbm, x0_hbm, xs1_hbm, macc_hbm,
                   a, b, cbuf, isd_v):
    base = _wid() * RPT
    pltpu.sync_copy(p_hbm.at[0, pl.ds(base, RPT)], a)
    pltpu.sync_copy(p_hbm.at[1, pl.ds(base, RPT)], b)
    pltpu.sync_copy(x0_hbm.at[pl.ds(base, RPT)], cbuf)
    pltpu.sync_copy(isd_hbm.at[pl.ds(base, RPT)], isd_v)

    def row(r, svec):
        for k in range(D // 16):
            sl = pl.ds(k * 16, 16)
            x1 = svec * (a[r, sl] + b[r, sl])
            cbuf[r, sl] = cbuf[r, sl] + x1
            a[r, sl] = svec * x1

    _scale_rows(isd_v, row)
    pltpu.sync_copy(a, xs1_hbm.at[pl.ds(base, RPT)])
    pltpu.sync_copy(cbuf, macc_hbm.at[pl.ds(base, RPT)])


def _final_body(p_hbm, isd_hbm, macc_hbm, out_hbm, a, b, cbuf, isd_v):
    base = _wid() * RPT
    pltpu.sync_copy(p_hbm.at[0, pl.ds(base, RPT)], a)
    pltpu.sync_copy(p_hbm.at[1, pl.ds(base, RPT)], b)
    pltpu.sync_copy(macc_hbm.at[pl.ds(base, RPT)], cbuf)
    pltpu.sync_copy(isd_hbm.at[pl.ds(base, RPT)], isd_v)
    third = jnp.full((16,), 1.0 / 3.0, _f32)

    def row(r, svec):
        sv3 = svec * third
        for k in range(D // 16):
            sl = pl.ds(k * 16, 16)
            cbuf[r, sl] = cbuf[r, sl] * third + sv3 * (a[r, sl] + b[r, sl])

    _scale_rows(isd_v, row)
    pltpu.sync_copy(cbuf, out_hbm.at[pl.ds(base, RPT)])


@functools.lru_cache(maxsize=None)
def _build():
    mesh = plsc.VectorSubcoreMesh(
        core_axis_name="c", subcore_axis_name="s",
        num_cores=NC, num_subcores=NS)
    sds = jax.ShapeDtypeStruct
    deg_k = pl.kernel(
        _deg_body, out_type=sds((NC, NPAD), _f32), mesh=mesh,
        scratch_types=[
            pltpu.VMEM((NCH, C), jnp.int32),
            pltpu.VMEM((C,), _f32),
            pltpu.VMEM_SHARED((NPAD,), _f32),
        ])
    edge_k = pl.kernel(
        _edge_body, out_type=sds((NC, NPAD, D), _f32), mesh=mesh,
        scratch_types=[
            pltpu.VMEM((NCB, C), jnp.int32),
            pltpu.VMEM((NCB, C), jnp.int32),
            pltpu.VMEM((C, D), _f32),
            pltpu.VMEM_SHARED((NPAD, D), _f32),
            pltpu.SemaphoreType.DMA,
        ])
    scale0_k = pl.kernel(
        _scale0_body, out_type=sds((NPAD, D), _f32), mesh=mesh,
        scratch_types=[
            pltpu.VMEM((RPT, D), _f32),
            pltpu.VMEM((RPT,), _f32),
        ])
    combine1_k = pl.kernel(
        _combine1_body,
        out_type=(sds((NPAD, D), _f32), sds((NPAD, D), _f32)), mesh=mesh,
        scratch_types=[
            pltpu.VMEM((RPT, D), _f32),
            pltpu.VMEM((RPT, D), _f32),
            pltpu.VMEM((RPT, D), _f32),
            pltpu.VMEM((RPT,), _f32),
        ])
    final_k = pl.kernel(
        _final_body, out_type=sds((NPAD, D), _f32), mesh=mesh,
        scratch_types=[
            pltpu.VMEM((RPT, D), _f32),
            pltpu.VMEM((RPT, D), _f32),
            pltpu.VMEM((RPT, D), _f32),
            pltpu.VMEM((RPT,), _f32),
        ])
    return deg_k, edge_k, scale0_k, combine1_k, final_k


def kernel(u_embs, i_embs, edge_index):
    deg_k, edge_k, scale0_k, combine1_k, final_k = _build()
    src = edge_index[0].astype(jnp.int32)
    dst = edge_index[1].astype(jnp.int32) + N_USERS
    rows = jnp.concatenate([src, dst])
    cols = jnp.concatenate([dst, src])
    pad = EPT - E2 // NW
    rows3 = jnp.pad(rows.reshape(NW, E2 // NW), ((0, 0), (0, pad)),
                    constant_values=TRASH).reshape(NW, NCH, C)
    cols3 = jnp.pad(cols.reshape(NW, E2 // NW), ((0, 0), (0, pad)),
                    constant_values=TRASH).reshape(NW, NCH, C)

    x0 = jnp.concatenate(
        [u_embs, i_embs, jnp.zeros((NPAD - N, D), _f32)], axis=0)
    zdeg = jnp.zeros((NPAD,), _f32)
    z2d = jnp.zeros((RPS, D), _f32)

    deg2 = deg_k(rows3, zdeg)
    isd = _isd_tc(deg2).reshape(NPAD)
    xs0 = scale0_k(x0, isd)
    p1 = edge_k(xs0, rows3, cols3, z2d)
    xs1, macc = combine1_k(p1, isd, x0)
    p2 = edge_k(xs1, rows3, cols3, z2d)
    out = final_k(p2, isd, macc)
    return out[:N]


# speedup vs baseline: 14.8523x; 1.1794x over previous
import functools

import jax
import jax.numpy as jnp
from jax import lax
from jax.experimental import pallas as pl
from jax.experimental.pallas import tpu as pltpu
from jax.experimental.pallas import tpu_sc as plsc

N_USERS = 5000
N_ITEMS = 5000
D = 128
HN = 5120
NPAD = 2 * HN
U_TRASH = 5100
I_TRASH = 10200
COL_TRASH = 5100
E = 320000
NC, NS = 2, 16
NW = NC * NS
EPT = 20480

CD = 128
NCHD = EPT // CD

C2 = 128
NCH2 = EPT // C2
NBUF = 4
NCB = 32
NBK = NCH2 // NCB

RPT = NPAD // NW
RPS = HN // NS

_f32 = jnp.float32


def _wid():
    return lax.axis_index("c") * NS + lax.axis_index("s")


def _deg_body(rows_hbm, zdeg_hbm, deg_hbm, rows_v, ones_v, deg_sh):
    c = lax.axis_index("c")
    s = lax.axis_index("s")
    @pl.when(s < 8)
    def _zero():
        pltpu.sync_copy(zdeg_hbm.at[pl.ds(s * 640, 640)],
                        deg_sh.at[pl.ds(s * 640, 640)])

    for k in range(CD // 16):
        ones_v[pl.ds(k * 16, 16)] = jnp.full((16,), 1.0, _f32)
    pltpu.sync_copy(rows_hbm.at[_wid()], rows_v)
    plsc.subcore_barrier()

    def step(j, carry):
        pltpu.sync_copy(ones_v, deg_sh.at[rows_v.at[j]], add=True)
        return carry

    lax.fori_loop(0, NCHD, step, 0)
    plsc.subcore_barrier()

    @pl.when(s < 8)
    def _drain():
        pltpu.sync_copy(deg_sh.at[pl.ds(s * 640, 640)],
                        deg_hbm.at[pl.ds(c * HN + s * 640, 640)])


def _isd_body(deg_ref, isd_ref):
    d = deg_ref[...]
    isd_ref[...] = jnp.where(
        d > 0.0, lax.rsqrt(jnp.maximum(d, 1.0)), 0.0)


def _isd_tc(deg):
    return pl.pallas_call(
        _isd_body,
        out_shape=jax.ShapeDtypeStruct((NPAD // D, D), _f32),
    )(deg.reshape(NPAD // D, D))


def _edge_body(xs_hbm, rows_hbm, cols_hbm, z2d_hbm, p_hbm,
               rows_v, cols_v, b0, b1, b2, b3, s0, s1, s2, s3, accum):
    bufs = [b0, b1, b2, b3]
    sems = [s0, s1, s2, s3]
    c = lax.axis_index("c")
    s = lax.axis_index("s")
    w = _wid()
    pltpu.sync_copy(z2d_hbm, accum.at[pl.ds(s * RPS, RPS)])
    plsc.subcore_barrier()

    def blk(bi, carry):
        pltpu.sync_copy(rows_hbm.at[w, pl.ds(bi * NCB, NCB)], rows_v)
        pltpu.sync_copy(cols_hbm.at[w, pl.ds(bi * NCB, NCB)], cols_v)

        def gather(j, k):
            pltpu.async_copy(xs_hbm.at[cols_v.at[j]], bufs[k], sems[k])

        def wait(j, k):
            pltpu.make_async_copy(
                xs_hbm.at[cols_v.at[j]], bufs[k], sems[k]).wait()

        def scat(j, k):
            pltpu.sync_copy(bufs[k], accum.at[rows_v.at[j]], add=True)

        for k in range(NBUF):
            gather(k, k)

        def grp(g, carry2):
            j0 = pl.multiple_of(g * NBUF, NBUF)
            for k in range(NBUF):
                wait(j0 + k, k)
                scat(j0 + k, k)
                gather(j0 + k + NBUF, k)
            return carry2

        lax.fori_loop(0, NCB // NBUF - 1, grp, 0)
        jl = NCB - NBUF
        for k in range(NBUF):
            wait(jl + k, k)
            scat(jl + k, k)
        return carry

    lax.fori_loop(0, NBK, blk, 0)
    plsc.subcore_barrier()
    pltpu.sync_copy(accum.at[pl.ds(s * RPS, RPS)],
                    p_hbm.at[pl.ds(c * HN + s * RPS, RPS)])


def _scale_rows(isd_v, fn):
    def grp(g, carry):
        base = pl.multiple_of(g * 16, 16)
        sv16 = isd_v[pl.ds(base, 16)]
        for i in range(16):
            fn(base + i, jnp.full((16,), sv16[i], _f32))
        return carry
    lax.fori_loop(0, RPT // 16, grp, 0)


def _scale0_body(x_hbm, isd_hbm, xs_hbm, buf, isd_v):
    base = _wid() * RPT
    pltpu.sync_copy(x_hbm.at[pl.ds(base, RPT)], buf)
    pltpu.sync_copy(isd_hbm.at[pl.ds(base, RPT)], isd_v)

    def row(r, svec):
        for k in range(D // 16):
            sl = pl.ds(k * 16, 16)
            buf[r, sl] = buf[r, sl] * svec

    _scale_rows(isd_v, row)
    pltpu.sync_copy(buf, xs_hbm.at[pl.ds(base, RPT)])


def _combine1_body(p_hbm, isd_hbm, x0_hbm, xs1_hbm, macc_hbm,
                   a, cbuf, isd_v):
    base = _wid() * RPT
    pltpu.sync_copy(p_hbm.at[pl.ds(base, RPT)], a)
    pltpu.sync_copy(x0_hbm.at[pl.ds(base, RPT)], cbuf)
    pltpu.sync_copy(isd_hbm.at[pl.ds(base, RPT)], isd_v)

    def row(r, svec):
        for k in range(D // 16):
            sl = pl.ds(k * 16, 16)
            x1 = svec * a[r, sl]
            cbuf[r, sl] = cbuf[r, sl] + x1
            a[r, sl] = svec * x1

    _scale_rows(isd_v, row)
    pltpu.sync_copy(a, xs1_hbm.at[pl.ds(base, RPT)])
    pltpu.sync_copy(cbuf, macc_hbm.at[pl.ds(base, RPT)])


def _final_body(p_hbm, isd_hbm, macc_hbm, out_hbm, a, cbuf, isd_v):
    base = _wid() * RPT
    pltpu.sync_copy(p_hbm.at[pl.ds(base, RPT)], a)
    pltpu.sync_copy(macc_hbm.at[pl.ds(base, RPT)], cbuf)
    pltpu.sync_copy(isd_hbm.at[pl.ds(base, RPT)], isd_v)
    third = jnp.full((16,), 1.0 / 3.0, _f32)

    def row(r, svec):
        sv3 = svec * third
        for k in range(D // 16):
            sl = pl.ds(k * 16, 16)
            cbuf[r, sl] = cbuf[r, sl] * third + sv3 * a[r, sl]

    _scale_rows(isd_v, row)
    pltpu.sync_copy(cbuf, out_hbm.at[pl.ds(base, RPT)])


@functools.lru_cache(maxsize=None)
def _build():
    mesh = plsc.VectorSubcoreMesh(
        core_axis_name="c", subcore_axis_name="s",
        num_cores=NC, num_subcores=NS)
    sds = jax.ShapeDtypeStruct
    deg_k = pl.kernel(
        _deg_body, out_type=sds((NPAD,), _f32), mesh=mesh,
        scratch_types=[
            pltpu.VMEM((NCHD, CD), jnp.int32),
            pltpu.VMEM((CD,), _f32),
            pltpu.VMEM_SHARED((HN,), _f32),
        ])
    edge_k = pl.kernel(
        _edge_body, out_type=sds((NPAD, D), _f32), mesh=mesh,
        scratch_types=[
            pltpu.VMEM((NCB, C2), jnp.int32),
            pltpu.VMEM((NCB, C2), jnp.int32),
            pltpu.VMEM((C2, D), _f32),
            pltpu.VMEM((C2, D), _f32),
            pltpu.VMEM((C2, D), _f32),
            pltpu.VMEM((C2, D), _f32),
            pltpu.SemaphoreType.DMA,
            pltpu.SemaphoreType.DMA,
            pltpu.SemaphoreType.DMA,
            pltpu.SemaphoreType.DMA,
            pltpu.VMEM_SHARED((HN, D), _f32),
        ])
    scale0_k = pl.kernel(
        _scale0_body, out_type=sds((NPAD, D), _f32), mesh=mesh,
        scratch_types=[
            pltpu.VMEM((RPT, D), _f32),
            pltpu.VMEM((RPT,), _f32),
        ])
    combine1_k = pl.kernel(
        _combine1_body,
        out_type=(sds((NPAD, D), _f32), sds((NPAD, D), _f32)), mesh=mesh,
        scratch_types=[
            pltpu.VMEM((RPT, D), _f32),
            pltpu.VMEM((RPT, D), _f32),
            pltpu.VMEM((RPT,), _f32),
        ])
    final_k = pl.kernel(
        _final_body, out_type=sds((NPAD, D), _f32), mesh=mesh,
        scratch_types=[
            pltpu.VMEM((RPT, D), _f32),
            pltpu.VMEM((RPT, D), _f32),
            pltpu.VMEM((RPT,), _f32),
        ])
    return deg_k, edge_k, scale0_k, combine1_k, final_k


def kernel(u_embs, i_embs, edge_index):
    deg_k, edge_k, scale0_k, combine1_k, final_k = _build()
    src = edge_index[0].astype(jnp.int32)
    dst = edge_index[1].astype(jnp.int32) + HN
    rows = jnp.concatenate([src, dst]).reshape(NW, 2 * E // NW)
    cols = jnp.concatenate([dst, src]).reshape(NW, 2 * E // NW)
    pad = EPT - 2 * E // NW
    is_item = (jnp.arange(NW, dtype=jnp.int32) >= NS)
    trash = jnp.where(is_item, I_TRASH, U_TRASH).astype(jnp.int32)
    rows = jnp.concatenate(
        [rows, jnp.broadcast_to(trash[:, None], (NW, pad))], axis=1)
    rows = rows - jnp.where(is_item, HN, 0).astype(jnp.int32)[:, None]
    cols = jnp.concatenate(
        [cols, jnp.full((NW, pad), COL_TRASH, jnp.int32)], axis=1)
    rows_d = rows.reshape(NW, NCHD, CD)
    rows_e = rows.reshape(NW, NCH2, C2)
    cols_e = cols.reshape(NW, NCH2, C2)

    zpad = jnp.zeros((HN - N_USERS, D), _f32)
    x0 = jnp.concatenate([u_embs, zpad, i_embs, zpad], axis=0)
    zdeg = jnp.zeros((HN,), _f32)
    z2d = jnp.zeros((RPS, D), _f32)

    deg = deg_k(rows_d, zdeg)
    isd = _isd_tc(deg).reshape(NPAD)
    xs0 = scale0_k(x0, isd)
    p1 = edge_k(xs0, rows_e, cols_e, z2d)
    xs1, macc = combine1_k(p1, isd, x0)
    p2 = edge_k(xs1, rows_e, cols_e, z2d)
    out = final_k(p2, isd, macc)
    return jnp.concatenate([out[:N_USERS], out[HN:HN + N_ITEMS]], axis=0)


# speedup vs baseline: 34.4631x; 2.3204x over previous
import functools

import jax
import jax.numpy as jnp
from jax import lax
from jax.experimental import pallas as pl
from jax.experimental.pallas import tpu as pltpu
from jax.experimental.pallas import tpu_sc as plsc

N_USERS = 5000
N_ITEMS = 5000
D = 128
HN = 5120
NPAD = 2 * HN
TRASH = 5080
E = 320000
NC, NS = 2, 16
NW = NC * NS
EPT = 20480

CD = 128
NCHD = EPT // CD

C2 = 128
NCH2 = EPT // C2
NBUF = 2
NCB = 32
NBK = NCH2 // NCB

RPT = NPAD // NW
RPS = HN // NS

_f32 = jnp.float32


def _wid():
    return lax.axis_index("c") * NS + lax.axis_index("s")


def _deg_body(rows_hbm, zdeg_hbm, deg_hbm, rows_v, ones_v, deg_sh):
    c = lax.axis_index("c")
    s = lax.axis_index("s")
    @pl.when(s < 8)
    def _zero():
        pltpu.sync_copy(zdeg_hbm.at[pl.ds(s * 640, 640)],
                        deg_sh.at[pl.ds(s * 640, 640)])

    for k in range(CD // 16):
        ones_v[pl.ds(k * 16, 16)] = jnp.full((16,), 1.0, _f32)
    pltpu.sync_copy(rows_hbm.at[_wid()], rows_v)
    plsc.subcore_barrier()

    def step(j, carry):
        pltpu.sync_copy(ones_v, deg_sh.at[rows_v.at[j]], add=True)
        return carry

    lax.fori_loop(0, NCHD, step, 0)
    plsc.subcore_barrier()

    @pl.when(s < 8)
    def _drain():
        pltpu.sync_copy(deg_sh.at[pl.ds(s * 640, 640)],
                        deg_hbm.at[pl.ds(c * HN + s * 640, 640)])


def _isd_body(deg_ref, isd_ref):
    d = deg_ref[...]
    isd_ref[...] = jnp.where(
        d > 0.0, lax.rsqrt(jnp.maximum(d, 1.0)), 0.0)


def _isd_tc(deg):
    return pl.pallas_call(
        _isd_body,
        out_shape=jax.ShapeDtypeStruct((NPAD // D, D), _f32),
    )(deg.reshape(NPAD // D, D))


def _edge_body(xs_hbm, rows_hbm, cols_hbm, z2d_hbm, p_hbm,
               rows_v, cols_v, b0, b1, s0, s1, xs_sh, accum):
    bufs = [b0, b1]
    sems = [s0, s1]
    c = lax.axis_index("c")
    s = lax.axis_index("s")
    w = _wid()
    pltpu.sync_copy(xs_hbm.at[pl.ds((1 - c) * HN + s * RPS, RPS)],
                    xs_sh.at[pl.ds(s * RPS, RPS)])
    pltpu.sync_copy(z2d_hbm, accum.at[pl.ds(s * RPS, RPS)])
    plsc.subcore_barrier()

    def blk(bi, carry):
        pltpu.sync_copy(rows_hbm.at[w, pl.ds(bi * NCB, NCB)], rows_v)
        pltpu.sync_copy(cols_hbm.at[w, pl.ds(bi * NCB, NCB)], cols_v)

        def gather(j, k):
            pltpu.async_copy(xs_sh.at[cols_v.at[j]], bufs[k], sems[k])

        def wait(j, k):
            pltpu.make_async_copy(
                xs_sh.at[cols_v.at[j]], bufs[k], sems[k]).wait()

        def scat(j, k):
            pltpu.sync_copy(bufs[k], accum.at[rows_v.at[j]], add=True)

        for k in range(NBUF):
            gather(k, k)

        def grp(g, carry2):
            j0 = pl.multiple_of(g * NBUF, NBUF)
            for k in range(NBUF):
                wait(j0 + k, k)
                scat(j0 + k, k)
                gather(j0 + k + NBUF, k)
            return carry2

        lax.fori_loop(0, NCB // NBUF - 1, grp, 0)
        jl = NCB - NBUF
        for k in range(NBUF):
            wait(jl + k, k)
            scat(jl + k, k)
        return carry

    lax.fori_loop(0, NBK, blk, 0)
    plsc.subcore_barrier()
    pltpu.sync_copy(accum.at[pl.ds(s * RPS, RPS)],
                    p_hbm.at[pl.ds(c * HN + s * RPS, RPS)])


def _scale_rows(isd_v, fn):
    def grp(g, carry):
        base = pl.multiple_of(g * 16, 16)
        sv16 = isd_v[pl.ds(base, 16)]
        for i in range(16):
            fn(base + i, jnp.full((16,), sv16[i], _f32))
        return carry
    lax.fori_loop(0, RPT // 16, grp, 0)


def _scale0_body(x_hbm, isd_hbm, xs_hbm, buf, isd_v):
    base = _wid() * RPT
    pltpu.sync_copy(x_hbm.at[pl.ds(base, RPT)], buf)
    pltpu.sync_copy(isd_hbm.at[pl.ds(base, RPT)], isd_v)

    def row(r, svec):
        for k in range(D // 16):
            sl = pl.ds(k * 16, 16)
            buf[r, sl] = buf[r, sl] * svec

    _scale_rows(isd_v, row)
    pltpu.sync_copy(buf, xs_hbm.at[pl.ds(base, RPT)])


def _combine1_body(p_hbm, isd_hbm, x0_hbm, xs1_hbm, macc_hbm,
                   a, cbuf, isd_v):
    base = _wid() * RPT
    pltpu.sync_copy(p_hbm.at[pl.ds(base, RPT)], a)
    pltpu.sync_copy(x0_hbm.at[pl.ds(base, RPT)], cbuf)
    pltpu.sync_copy(isd_hbm.at[pl.ds(base, RPT)], isd_v)

    def row(r, svec):
        for k in range(D // 16):
            sl = pl.ds(k * 16, 16)
            x1 = svec * a[r, sl]
            cbuf[r, sl] = cbuf[r, sl] + x1
            a[r, sl] = svec * x1

    _scale_rows(isd_v, row)
    pltpu.sync_copy(a, xs1_hbm.at[pl.ds(base, RPT)])
    pltpu.sync_copy(cbuf, macc_hbm.at[pl.ds(base, RPT)])


def _final_body(p_hbm, isd_hbm, macc_hbm, out_hbm, a, cbuf, isd_v):
    base = _wid() * RPT
    pltpu.sync_copy(p_hbm.at[pl.ds(base, RPT)], a)
    pltpu.sync_copy(macc_hbm.at[pl.ds(base, RPT)], cbuf)
    pltpu.sync_copy(isd_hbm.at[pl.ds(base, RPT)], isd_v)
    third = jnp.full((16,), 1.0 / 3.0, _f32)

    def row(r, svec):
        sv3 = svec * third
        for k in range(D // 16):
            sl = pl.ds(k * 16, 16)
            cbuf[r, sl] = cbuf[r, sl] * third + sv3 * a[r, sl]

    _scale_rows(isd_v, row)
    pltpu.sync_copy(cbuf, out_hbm.at[pl.ds(base, RPT)])


@functools.lru_cache(maxsize=None)
def _build():
    mesh = plsc.VectorSubcoreMesh(
        core_axis_name="c", subcore_axis_name="s",
        num_cores=NC, num_subcores=NS)
    sds = jax.ShapeDtypeStruct
    deg_k = pl.kernel(
        _deg_body, out_type=sds((NPAD,), _f32), mesh=mesh,
        scratch_types=[
            pltpu.VMEM((NCHD, CD), jnp.int32),
            pltpu.VMEM((CD,), _f32),
            pltpu.VMEM_SHARED((HN,), _f32),
        ])
    edge_k = pl.kernel(
        _edge_body, out_type=sds((NPAD, D), _f32), mesh=mesh,
        scratch_types=[
            pltpu.VMEM((NCB, C2), jnp.int32),
            pltpu.VMEM((NCB, C2), jnp.int32),
            pltpu.VMEM((C2, D), _f32),
            pltpu.VMEM((C2, D), _f32),
            pltpu.SemaphoreType.DMA,
            pltpu.SemaphoreType.DMA,
            pltpu.VMEM_SHARED((HN, D), _f32),
            pltpu.VMEM_SHARED((HN, D), _f32),
        ])
    scale0_k = pl.kernel(
        _scale0_body, out_type=sds((NPAD, D), _f32), mesh=mesh,
        scratch_types=[
            pltpu.VMEM((RPT, D), _f32),
            pltpu.VMEM((RPT,), _f32),
        ])
    combine1_k = pl.kernel(
        _combine1_body,
        out_type=(sds((NPAD, D), _f32), sds((NPAD, D), _f32)), mesh=mesh,
        scratch_types=[
            pltpu.VMEM((RPT, D), _f32),
            pltpu.VMEM((RPT, D), _f32),
            pltpu.VMEM((RPT,), _f32),
        ])
    final_k = pl.kernel(
        _final_body, out_type=sds((NPAD, D), _f32), mesh=mesh,
        scratch_types=[
            pltpu.VMEM((RPT, D), _f32),
            pltpu.VMEM((RPT, D), _f32),
            pltpu.VMEM((RPT,), _f32),
        ])
    return deg_k, edge_k, scale0_k, combine1_k, final_k


def kernel(u_embs, i_embs, edge_index):
    deg_k, edge_k, scale0_k, combine1_k, final_k = _build()
    src = edge_index[0].astype(jnp.int32)
    dst = edge_index[1].astype(jnp.int32)
    rows = jnp.concatenate([src, dst]).reshape(NW, 2 * E // NW)
    cols = jnp.concatenate([dst, src]).reshape(NW, 2 * E // NW)
    pad = EPT - 2 * E // NW
    rows = jnp.concatenate(
        [rows, jnp.full((NW, pad), TRASH, jnp.int32)], axis=1)
    cols = jnp.concatenate(
        [cols, jnp.full((NW, pad), TRASH, jnp.int32)], axis=1)
    rows_d = rows.reshape(NW, NCHD, CD)
    rows_e = rows.reshape(NW, NCH2, C2)
    cols_e = cols.reshape(NW, NCH2, C2)

    zpad = jnp.zeros((HN - N_USERS, D), _f32)
    x0 = jnp.concatenate([u_embs, zpad, i_embs, zpad], axis=0)
    zdeg = jnp.zeros((HN,), _f32)
    z2d = jnp.zeros((RPS, D), _f32)

    deg = deg_k(rows_d, zdeg)
    isd = _isd_tc(deg).reshape(NPAD)
    xs0 = scale0_k(x0, isd)
    p1 = edge_k(xs0, rows_e, cols_e, z2d)
    xs1, macc = combine1_k(p1, isd, x0)
    p2 = edge_k(xs1, rows_e, cols_e, z2d)
    out = final_k(p2, isd, macc)
    return jnp.concatenate([out[:N_USERS], out[HN:HN + N_ITEMS]], axis=0)


# speedup vs baseline: 34.5614x; 1.0029x over previous
import functools

import jax
import jax.numpy as jnp
from jax import lax
from jax.experimental import pallas as pl
from jax.experimental.pallas import tpu as pltpu
from jax.experimental.pallas import tpu_sc as plsc

N_USERS = 5000
N_ITEMS = 5000
D = 128
HN = 5120
NPAD = 2 * HN
TRASH = 5080
E = 320000
NC, NS = 2, 16
NW = NC * NS
EPT = 20480

CD = 128
NCHD = EPT // CD

C2 = 128
NCH2 = EPT // C2
NBUF = 2
NCB = 32
NBK = NCH2 // NCB

RPT = NPAD // NW
RPS = HN // NS

_f32 = jnp.float32


def _wid():
    return lax.axis_index("c") * NS + lax.axis_index("s")


def _deg_body(rows_hbm, zdeg_hbm, deg_hbm, rows_v, ones_v, deg_sh):
    c = lax.axis_index("c")
    s = lax.axis_index("s")
    @pl.when(s < 8)
    def _zero():
        pltpu.sync_copy(zdeg_hbm.at[pl.ds(s * 640, 640)],
                        deg_sh.at[pl.ds(s * 640, 640)])

    for k in range(CD // 16):
        ones_v[pl.ds(k * 16, 16)] = jnp.full((16,), 1.0, _f32)
    pltpu.sync_copy(rows_hbm.at[_wid()], rows_v)
    plsc.subcore_barrier()

    def step(j, carry):
        pltpu.sync_copy(ones_v, deg_sh.at[rows_v.at[j]], add=True)
        return carry

    lax.fori_loop(0, NCHD, step, 0)
    plsc.subcore_barrier()

    @pl.when(s < 8)
    def _drain():
        pltpu.sync_copy(deg_sh.at[pl.ds(s * 640, 640)],
                        deg_hbm.at[pl.ds(c * HN + s * 640, 640)])


def _isd_body(deg_ref, isd_ref):
    d = deg_ref[...]
    isd_ref[...] = jnp.where(
        d > 0.0, lax.rsqrt(jnp.maximum(d, 1.0)), 0.0)


def _isd_tc(deg):
    return pl.pallas_call(
        _isd_body,
        out_shape=jax.ShapeDtypeStruct((NPAD // D, D), _f32),
    )(deg.reshape(NPAD // D, D))


def _edge_loop(rows_hbm, cols_hbm, p_hbm, rows_v, cols_v, bufs, sems,
               xs_sh, accum, c, s, w):
    def blk(bi, carry):
        pltpu.sync_copy(rows_hbm.at[w, pl.ds(bi * NCB, NCB)], rows_v)
        pltpu.sync_copy(cols_hbm.at[w, pl.ds(bi * NCB, NCB)], cols_v)

        def gather(j, k):
            pltpu.async_copy(xs_sh.at[cols_v.at[j]], bufs[k], sems[k])

        def wait(j, k):
            pltpu.make_async_copy(
                xs_sh.at[cols_v.at[j]], bufs[k], sems[k]).wait()

        def scat(j, k):
            pltpu.sync_copy(bufs[k], accum.at[rows_v.at[j]], add=True)

        for k in range(NBUF):
            gather(k, k)

        def grp(g, carry2):
            j0 = pl.multiple_of(g * NBUF, NBUF)
            for k in range(NBUF):
                wait(j0 + k, k)
                scat(j0 + k, k)
                gather(j0 + k + NBUF, k)
            return carry2

        lax.fori_loop(0, NCB // NBUF - 1, grp, 0)
        jl = NCB - NBUF
        for k in range(NBUF):
            wait(jl + k, k)
            scat(jl + k, k)
        return carry

    lax.fori_loop(0, NBK, blk, 0)
    plsc.subcore_barrier()
    pltpu.sync_copy(accum.at[pl.ds(s * RPS, RPS)],
                    p_hbm.at[pl.ds(c * HN + s * RPS, RPS)])


_SUB = ((0, 128), (128, 128), (256, 64))


def _opp_scale_rows(isd_v, off, n, fn):
    def grp(g, carry):
        base = off + pl.multiple_of(g * 16, 16)
        sv16 = isd_v[pl.ds(base, 16)]
        for i in range(16):
            fn(base + i, jnp.full((16,), sv16[i], _f32))
        return carry
    lax.fori_loop(0, n // 16, grp, 0)


def _edge1_body(x0_hbm, isd_hbm, rows_hbm, cols_hbm, z2d_hbm, p_hbm,
                rows_v, cols_v, b0, b1, s0, s1, isd_v, xs_sh, accum):
    c = lax.axis_index("c")
    s = lax.axis_index("s")
    w = _wid()
    ob = (1 - c) * HN + s * RPS
    pltpu.sync_copy(z2d_hbm, accum.at[pl.ds(s * RPS, RPS)])
    pltpu.sync_copy(isd_hbm.at[pl.ds(ob, RPS)], isd_v)
    for off, n in _SUB:
        pltpu.sync_copy(x0_hbm.at[pl.ds(ob + off, n)], b0.at[pl.ds(0, n)])

        def row(r, svec):
            rr = r - off
            for k in range(D // 16):
                sl = pl.ds(k * 16, 16)
                b0[rr, sl] = b0[rr, sl] * svec

        _opp_scale_rows(isd_v, off, n, row)
        pltpu.sync_copy(b0.at[pl.ds(0, n)],
                        xs_sh.at[pl.ds(s * RPS + off, n)])
    plsc.subcore_barrier()
    _edge_loop(rows_hbm, cols_hbm, p_hbm, rows_v, cols_v,
               [b0, b1], [s0, s1], xs_sh, accum, c, s, w)


def _edge2_body(p1_hbm, isd_hbm, x0_hbm, rows_hbm, cols_hbm, z2d_hbm,
                p_hbm, macc_hbm,
                rows_v, cols_v, b0, b1, s0, s1, isd_v, xs_sh, accum):
    c = lax.axis_index("c")
    s = lax.axis_index("s")
    w = _wid()
    ob = (1 - c) * HN + s * RPS
    pltpu.sync_copy(z2d_hbm, accum.at[pl.ds(s * RPS, RPS)])
    pltpu.sync_copy(isd_hbm.at[pl.ds(ob, RPS)], isd_v)
    for off, n in _SUB:
        pltpu.sync_copy(p1_hbm.at[pl.ds(ob + off, n)], b0.at[pl.ds(0, n)])
        pltpu.sync_copy(x0_hbm.at[pl.ds(ob + off, n)], b1.at[pl.ds(0, n)])

        def row(r, svec):
            rr = r - off
            for k in range(D // 16):
                sl = pl.ds(k * 16, 16)
                x1 = svec * b0[rr, sl]
                b1[rr, sl] = b1[rr, sl] + x1
                b0[rr, sl] = svec * x1

        _opp_scale_rows(isd_v, off, n, row)
        pltpu.sync_copy(b0.at[pl.ds(0, n)],
                        xs_sh.at[pl.ds(s * RPS + off, n)])
        pltpu.sync_copy(b1.at[pl.ds(0, n)],
                        macc_hbm.at[pl.ds(ob + off, n)])
    plsc.subcore_barrier()
    _edge_loop(rows_hbm, cols_hbm, p_hbm, rows_v, cols_v,
               [b0, b1], [s0, s1], xs_sh, accum, c, s, w)


def _scale_rows(isd_v, fn):
    def grp(g, carry):
        base = pl.multiple_of(g * 16, 16)
        sv16 = isd_v[pl.ds(base, 16)]
        for i in range(16):
            fn(base + i, jnp.full((16,), sv16[i], _f32))
        return carry
    lax.fori_loop(0, RPT // 16, grp, 0)


def _final_body(p_hbm, isd_hbm, macc_hbm, out_hbm, a, cbuf, isd_v):
    base = _wid() * RPT
    pltpu.sync_copy(p_hbm.at[pl.ds(base, RPT)], a)
    pltpu.sync_copy(macc_hbm.at[pl.ds(base, RPT)], cbuf)
    pltpu.sync_copy(isd_hbm.at[pl.ds(base, RPT)], isd_v)
    third = jnp.full((16,), 1.0 / 3.0, _f32)

    def row(r, svec):
        sv3 = svec * third
        for k in range(D // 16):
            sl = pl.ds(k * 16, 16)
            cbuf[r, sl] = cbuf[r, sl] * third + sv3 * a[r, sl]

    _scale_rows(isd_v, row)
    pltpu.sync_copy(cbuf, out_hbm.at[pl.ds(base, RPT)])


@functools.lru_cache(maxsize=None)
def _build():
    mesh = plsc.VectorSubcoreMesh(
        core_axis_name="c", subcore_axis_name="s",
        num_cores=NC, num_subcores=NS)
    sds = jax.ShapeDtypeStruct
    deg_k = pl.kernel(
        _deg_body, out_type=sds((NPAD,), _f32), mesh=mesh,
        scratch_types=[
            pltpu.VMEM((NCHD, CD), jnp.int32),
            pltpu.VMEM((CD,), _f32),
            pltpu.VMEM_SHARED((HN,), _f32),
        ])
    edge_scratch = [
        pltpu.VMEM((NCB, C2), jnp.int32),
        pltpu.VMEM((NCB, C2), jnp.int32),
        pltpu.VMEM((C2, D), _f32),
        pltpu.VMEM((C2, D), _f32),
        pltpu.SemaphoreType.DMA,
        pltpu.SemaphoreType.DMA,
        pltpu.VMEM((RPS,), _f32),
        pltpu.VMEM_SHARED((HN, D), _f32),
        pltpu.VMEM_SHARED((HN, D), _f32),
    ]
    edge1_k = pl.kernel(
        _edge1_body, out_type=sds((NPAD, D), _f32), mesh=mesh,
        scratch_types=list(edge_scratch))
    edge2_k = pl.kernel(
        _edge2_body,
        out_type=(sds((NPAD, D), _f32), sds((NPAD, D), _f32)), mesh=mesh,
        scratch_types=list(edge_scratch))
    final_k = pl.kernel(
        _final_body, out_type=sds((NPAD, D), _f32), mesh=mesh,
        scratch_types=[
            pltpu.VMEM((RPT, D), _f32),
            pltpu.VMEM((RPT, D), _f32),
            pltpu.VMEM((RPT,), _f32),
        ])
    return deg_k, edge1_k, edge2_k, final_k


def kernel(u_embs, i_embs, edge_index):
    deg_k, edge1_k, edge2_k, final_k = _build()
    src = edge_index[0].astype(jnp.int32)
    dst = edge_index[1].astype(jnp.int32)
    rows = jnp.concatenate([src, dst]).reshape(NW, 2 * E // NW)
    cols = jnp.concatenate([dst, src]).reshape(NW, 2 * E // NW)
    pad = EPT - 2 * E // NW
    rows = jnp.concatenate(
        [rows, jnp.full((NW, pad), TRASH, jnp.int32)], axis=1)
    cols = jnp.concatenate(
        [cols, jnp.full((NW, pad), TRASH, jnp.int32)], axis=1)
    rows_d = rows.reshape(NW, NCHD, CD)
    rows_e = rows.reshape(NW, NCH2, C2)
    cols_e = cols.reshape(NW, NCH2, C2)

    zpad = jnp.zeros((HN - N_USERS, D), _f32)
    x0 = jnp.concatenate([u_embs, zpad, i_embs, zpad], axis=0)
    zdeg = jnp.zeros((HN,), _f32)
    z2d = jnp.zeros((RPS, D), _f32)

    deg = deg_k(rows_d, zdeg)
    isd = _isd_tc(deg).reshape(NPAD)
    p1 = edge1_k(x0, isd, rows_e, cols_e, z2d)
    p2, macc = edge2_k(p1, isd, x0, rows_e, cols_e, z2d)
    out = final_k(p2, isd, macc)
    return jnp.concatenate([out[:N_USERS], out[HN:HN + N_ITEMS]], axis=0)
